# baseline (speedup 1.0000x reference)
import functools

import jax
import jax.numpy as jnp
from jax.experimental import pallas as pl
from jax.experimental.pallas import tpu as pltpu

ALPHA = 0.01


def _bl_kernel(x_ref, counts_ref, sums_ref, *, K):
    s = pl.program_id(1)

    @pl.when(s == 0)
    def _init():
        counts_ref[...] = jnp.zeros_like(counts_ref)
        sums_ref[...] = jnp.zeros_like(sums_ref)

    x_orig = x_ref[0]
    sums_ref[0] += jnp.sum(x_orig, axis=0, keepdims=True)

    x = x_orig.T
    thresh = jnp.max(x, axis=0, keepdims=True)
    for _ in range(K - 1):
        thresh = jnp.max(jnp.where(x < thresh, x, -jnp.inf), axis=0, keepdims=True)
    sel = (x >= thresh).astype(jnp.float32)
    counts_ref[0] += jnp.sum(sel, axis=1, keepdims=True).T


def _contract_kernel(counts_ref, sums_ref, loss_ref):
    loss_ref[...] = jnp.sum(counts_ref[...] * sums_ref[...]).reshape(1, 1)


def kernel(router_weights, n_routed_experts, num_experts_per_tok, router_n_groups):
    rw = router_weights.astype(jnp.float32)
    L, S, E = rw.shape
    K = 8
    T = min(8192, S)
    NS = S // T

    counts, sums = pl.pallas_call(
        functools.partial(_bl_kernel, K=K),
        grid=(L, NS),
        in_specs=[pl.BlockSpec((1, T, E), lambda l, s: (l, s, 0))],
        out_specs=[
            pl.BlockSpec((1, 1, E), lambda l, s: (l, 0, 0)),
            pl.BlockSpec((1, 1, E), lambda l, s: (l, 0, 0)),
        ],
        out_shape=[
            jax.ShapeDtypeStruct((L, 1, E), jnp.float32),
            jax.ShapeDtypeStruct((L, 1, E), jnp.float32),
        ],
        compiler_params=pltpu.CompilerParams(
            dimension_semantics=("parallel", "arbitrary"),
        ),
    )(rw)

    loss = pl.pallas_call(
        _contract_kernel,
        out_shape=jax.ShapeDtypeStruct((1, 1), jnp.float32),
    )(counts, sums)

    scale = n_routed_experts / (S * num_experts_per_tok)
    return loss[0, 0] * scale * (ALPHA / S)


# speedup vs baseline: 1.2496x; 1.2496x over previous
import functools

import jax
import jax.numpy as jnp
from jax.experimental import pallas as pl
from jax.experimental.pallas import tpu as pltpu

ALPHA = 0.01


def _bl_kernel(x_ref, counts_ref, sums_ref, *, K, NS):
    s = pl.program_id(1)

    @pl.when(s == 0)
    def _init():
        counts_ref[...] = jnp.zeros_like(counts_ref)
        sums_ref[...] = jnp.zeros_like(sums_ref)

    x_orig = x_ref[...]
    sums_ref[0] += jnp.sum(x_orig, axis=0, keepdims=True)

    x = x_orig.T
    thresh = jnp.max(x, axis=0, keepdims=True)
    for _ in range(K - 1):
        thresh = jnp.max(jnp.where(x < thresh, x, -jnp.inf), axis=0, keepdims=True)
    sel = (x >= thresh).astype(jnp.float32)
    counts_ref[0] += jnp.sum(sel, axis=1, keepdims=True).T


def _contract_kernel(counts_ref, sums_ref, loss_ref):
    loss_ref[...] = jnp.sum(counts_ref[...] * sums_ref[...]).reshape(1, 1)


def kernel(router_weights, n_routed_experts, num_experts_per_tok, router_n_groups):
    L, S, E = router_weights.shape
    K = 8
    T = min(8192, S)
    NS = S // T
    rw = router_weights.reshape(L * S, E)

    counts, sums = pl.pallas_call(
        functools.partial(_bl_kernel, K=K, NS=NS),
        grid=(L, NS),
        in_specs=[pl.BlockSpec((T, E), lambda l, s: (l * NS + s, 0))],
        out_specs=[
            pl.BlockSpec((1, 1, E), lambda l, s: (l, 0, 0)),
            pl.BlockSpec((1, 1, E), lambda l, s: (l, 0, 0)),
        ],
        out_shape=[
            jax.ShapeDtypeStruct((L, 1, E), jnp.float32),
            jax.ShapeDtypeStruct((L, 1, E), jnp.float32),
        ],
        compiler_params=pltpu.CompilerParams(
            dimension_semantics=("parallel", "arbitrary"),
        ),
    )(rw)

    loss = pl.pallas_call(
        _contract_kernel,
        out_shape=jax.ShapeDtypeStruct((1, 1), jnp.float32),
    )(counts, sums)

    scale = n_routed_experts / (S * num_experts_per_tok)
    return loss[0, 0] * scale * (ALPHA / S)


# speedup vs baseline: 1.2537x; 1.0033x over previous
import functools

import jax
import jax.numpy as jnp
from jax.experimental import pallas as pl
from jax.experimental.pallas import tpu as pltpu

ALPHA = 0.01


def _bl_kernel(x_ref, loss_ref, counts_ref, sums_ref, *, L, NB, K):
    i = pl.program_id(0)
    l = i // NB

    @pl.when(i == 0)
    def _init():
        counts_ref[...] = jnp.zeros_like(counts_ref)
        sums_ref[...] = jnp.zeros_like(sums_ref)

    x_orig = x_ref[...]
    sums_ref[pl.ds(l, 1), :] += jnp.sum(x_orig, axis=0, keepdims=True)

    x = x_orig.T
    thresh = jnp.max(x, axis=0, keepdims=True)
    for _ in range(K - 1):
        thresh = jnp.max(jnp.where(x < thresh, x, -jnp.inf), axis=0, keepdims=True)
    sel = (x >= thresh).astype(jnp.float32)
    counts_ref[pl.ds(l, 1), :] += jnp.sum(sel, axis=1, keepdims=True).T

    @pl.when(i == L * NB - 1)
    def _fin():
        loss_ref[...] = jnp.sum(counts_ref[...] * sums_ref[...]).reshape(1, 1)


def kernel(router_weights, n_routed_experts, num_experts_per_tok, router_n_groups):
    L, S, E = router_weights.shape
    K = 8
    T = min(8192, S)
    NB = S // T
    rw = router_weights.reshape(L * S, E)

    out = pl.pallas_call(
        functools.partial(_bl_kernel, L=L, NB=NB, K=K),
        grid=(L * NB,),
        in_specs=[pl.BlockSpec((T, E), lambda i: (i, 0))],
        out_specs=pl.BlockSpec((1, 1), lambda i: (0, 0)),
        out_shape=jax.ShapeDtypeStruct((1, 1), jnp.float32),
        scratch_shapes=[
            pltpu.VMEM((L, E), jnp.float32),
            pltpu.VMEM((L, E), jnp.float32),
        ],
    )(rw)

    scale = n_routed_experts / (S * num_experts_per_tok)
    return out[0, 0] * scale * (ALPHA / S)
